# baseline (speedup 1.0000x reference)
import functools

import jax
import jax.numpy as jnp
from jax import lax
from jax.experimental import pallas as pl
from jax.experimental.pallas import tpu as pltpu
from jax.experimental.pallas import tpu_sc as plsc

PUSH_MARGIN = 1.0
NUM_CLASSES = 8
B, N, C, H, W = 4, 4096, 32, 512, 512
PTS = B * N
NC, NS, L = 2, 16, 16
NW = NC * NS
PER_W = PTS // NW
CHW = 128
NCH = PER_W // CHW


@functools.cache
def _gather_labels_kernel():
    mesh = plsc.VectorSubcoreMesh(
        core_axis_name="c", subcore_axis_name="s", num_cores=NC, num_subcores=NS
    )

    @functools.partial(
        pl.kernel,
        out_type=jax.ShapeDtypeStruct((B, N), jnp.int32),
        mesh=mesh,
        scratch_types=[
            pltpu.VMEM((PER_W,), jnp.int32),
            pltpu.VMEM((PER_W,), jnp.int32),
            pltpu.VMEM((NCH, CHW), jnp.int32),
            pltpu.VMEM((PER_W,), jnp.int32),
            pltpu.SemaphoreType.DMA,
        ],
    )
    def _gather_labels(coords_hbm, y_hbm, out_hbm, cy_v, cx_v, idx_v, lab_v, sem):
        wid = lax.axis_index("s") * NC + lax.axis_index("c")
        base = wid * PER_W
        boff = (base // N) * (H * W)
        cpy = pltpu.async_copy(coords_hbm.at[pl.ds(base, PER_W)], cy_v, sem)
        cpx = pltpu.async_copy(coords_hbm.at[pl.ds(PTS + base, PER_W)], cx_v, sem)
        cpy.wait()
        cpx.wait()
        for j in range(NCH):
            for t in range(CHW // L):
                s = pl.ds(j * CHW + t * L, L)
                idx_v[j, pl.ds(t * L, L)] = cx_v[s] * W + cy_v[s] + boff
        copies = [
            pltpu.async_copy(y_hbm.at[idx_v.at[j]], lab_v.at[pl.ds(j * CHW, CHW)], sem)
            for j in range(NCH)
        ]
        for cp in copies:
            cp.wait()
        pltpu.sync_copy(lab_v, out_hbm.at[base // N, pl.ds(base % N, PER_W)])

    return _gather_labels


def _loss_kernel(embt_ref, lab_ref, out_ref):
    dn_ss = (((1,), (1,)), ((), ()))
    dn_mm = (((1,), (0,)), ((), ()))
    prec = lax.Precision.DEFAULT
    cls_col = lax.broadcasted_iota(jnp.int32, (NUM_CLASSES, 1), 0)
    fg_col = (cls_col >= 1).astype(jnp.float32)
    rr = lax.broadcasted_iota(jnp.int32, (NUM_CLASSES, NUM_CLASSES), 0)
    cc = lax.broadcasted_iota(jnp.int32, (NUM_CLASSES, NUM_CLASSES), 1)
    upper = ((cc > rr) & (rr >= 1)).astype(jnp.float32)

    total = jnp.float32(0.0)
    for b in range(B):
        embt = lax.transpose(embt_ref[b], (1, 0))
        lab = lab_ref[b].reshape(1, N)
        oh = (lab == cls_col).astype(jnp.float32)
        counts_col = jnp.sum(oh, axis=1, keepdims=True)
        safe_col = jnp.maximum(counts_col, 1.0)
        safe_row = lax.transpose(safe_col, (1, 0))
        sums_t = lax.dot_general(embt, oh, dn_ss, precision=prec,
                                 preferred_element_type=jnp.float32)
        cents_t = sums_t / safe_row
        cpp_t = lax.dot_general(cents_t, oh, dn_mm, precision=prec,
                                preferred_element_type=jnp.float32)
        diff = embt - cpp_t
        d = jnp.sqrt(jnp.sum(diff * diff, axis=0, keepdims=True))
        pull_sums = jnp.sum(oh * d, axis=1, keepdims=True)
        pull_c = pull_sums / safe_col
        presf_col = (counts_col > 0.0).astype(jnp.float32) * fg_col
        pull_over = jnp.sum(pull_c * presf_col)
        k = jnp.sum(presf_col)

        gram = lax.dot_general(cents_t, cents_t, (((0,), (0,)), ((), ())),
                               precision=lax.Precision.HIGHEST,
                               preferred_element_type=jnp.float32)
        n2_row = jnp.sum(cents_t * cents_t, axis=0, keepdims=True)
        n2_col = lax.transpose(n2_row, (1, 0))
        pd2 = jnp.maximum(n2_row + n2_col - 2.0 * gram, 0.0)
        pd = jnp.sqrt(pd2)
        presf_row = lax.transpose(presf_col, (1, 0))
        pairm = presf_col * presf_row * upper
        n_pairs = jnp.sum(pairm)
        push_sum = jnp.sum(jnp.maximum(PUSH_MARGIN - pd, 0.0) * pairm)
        push_term = push_sum / jnp.maximum(n_pairs, 1.0)

        multi = k > 1.0
        contrib = pull_over / jnp.maximum(k, 1.0) + push_term
        total = total + jnp.where(multi, contrib, 0.0)
    out_ref[0, 0] = total


def _loss_from_labels(embt, labels):
    return pl.pallas_call(
        _loss_kernel,
        out_shape=jax.ShapeDtypeStruct((1, 1), jnp.float32),
        out_specs=pl.BlockSpec(memory_space=pltpu.SMEM),
    )(embt, labels)


def kernel(abs_embedding, coordinates, y):
    coords_t = jnp.transpose(coordinates.astype(jnp.int32), (2, 0, 1)).reshape(-1)
    y_flat = y.reshape(-1).astype(jnp.int32)
    labels = _gather_labels_kernel()(coords_t, y_flat)
    loss = _loss_from_labels(abs_embedding, labels)
    return loss[0, 0]


# speedup vs baseline: 1.0362x; 1.0362x over previous
import functools

import jax
import jax.numpy as jnp
from jax import lax
from jax.experimental import pallas as pl
from jax.experimental.pallas import tpu as pltpu
from jax.experimental.pallas import tpu_sc as plsc

PUSH_MARGIN = 1.0
NUM_CLASSES = 8
B, N, C, H, W = 4, 4096, 32, 512, 512
PTS = B * N
NC, NS, L = 2, 16, 16
NW = NC * NS
PER_W = PTS // NW
CHW = 128
NCH = PER_W // CHW


@functools.cache
def _gather_labels_kernel():
    mesh = plsc.VectorSubcoreMesh(
        core_axis_name="c", subcore_axis_name="s", num_cores=NC, num_subcores=NS
    )

    @functools.partial(
        pl.kernel,
        out_type=jax.ShapeDtypeStruct((B, N), jnp.int32),
        mesh=mesh,
        scratch_types=[
            pltpu.VMEM((PER_W,), jnp.int32),
            pltpu.VMEM((PER_W,), jnp.int32),
            pltpu.VMEM((NCH, CHW), jnp.int32),
            pltpu.VMEM((PER_W,), jnp.int32),
            pltpu.SemaphoreType.DMA,
        ],
    )
    def _gather_labels(coords_hbm, y_hbm, out_hbm, cy_v, cx_v, idx_v, lab_v, sem):
        wid = lax.axis_index("s") * NC + lax.axis_index("c")
        base = wid * PER_W
        boff = (base // N) * (H * W)
        cpy = pltpu.async_copy(coords_hbm.at[pl.ds(base, PER_W)], cy_v, sem)
        cpx = pltpu.async_copy(coords_hbm.at[pl.ds(PTS + base, PER_W)], cx_v, sem)
        cpy.wait()
        cpx.wait()
        for j in range(NCH):
            for t in range(CHW // L):
                s = pl.ds(j * CHW + t * L, L)
                idx_v[j, pl.ds(t * L, L)] = cx_v[s] * W + cy_v[s] + boff
        copies = [
            pltpu.async_copy(y_hbm.at[idx_v.at[j]], lab_v.at[pl.ds(j * CHW, CHW)], sem)
            for j in range(NCH)
        ]
        for cp in copies:
            cp.wait()
        pltpu.sync_copy(lab_v, out_hbm.at[base // N, pl.ds(base % N, PER_W)])

    return _gather_labels


def _loss_kernel(embt_ref, lab_ref, out_ref):
    dn_ss = (((1,), (1,)), ((), ()))
    dn_mm = (((1,), (0,)), ((), ()))
    prec = lax.Precision.DEFAULT
    cls_col = lax.broadcasted_iota(jnp.int32, (NUM_CLASSES, 1), 0)
    fg_col = (cls_col >= 1).astype(jnp.float32)
    rr = lax.broadcasted_iota(jnp.int32, (NUM_CLASSES, NUM_CLASSES), 0)
    cc = lax.broadcasted_iota(jnp.int32, (NUM_CLASSES, NUM_CLASSES), 1)
    upper = ((cc > rr) & (rr >= 1)).astype(jnp.float32)

    eye = (lax.broadcasted_iota(jnp.int32, (C, C), 0)
           == lax.broadcasted_iota(jnp.int32, (C, C), 1)).astype(jnp.float32)
    total = jnp.float32(0.0)
    for b in range(B):
        embt = lax.dot_general(eye, embt_ref[b], dn_ss, precision=prec,
                               preferred_element_type=jnp.float32)
        lab = lab_ref[b].reshape(1, N)
        oh = (lab == cls_col).astype(jnp.float32)
        counts_col = jnp.sum(oh, axis=1, keepdims=True)
        safe_col = jnp.maximum(counts_col, 1.0)
        safe_row = lax.transpose(safe_col, (1, 0))
        sums_t = lax.dot_general(embt, oh, dn_ss, precision=prec,
                                 preferred_element_type=jnp.float32)
        cents_t = sums_t / safe_row
        cpp_t = lax.dot_general(cents_t, oh, dn_mm, precision=prec,
                                preferred_element_type=jnp.float32)
        diff = embt - cpp_t
        d = jnp.sqrt(jnp.sum(diff * diff, axis=0, keepdims=True))
        pull_sums = jnp.sum(oh * d, axis=1, keepdims=True)
        pull_c = pull_sums / safe_col
        presf_col = (counts_col > 0.0).astype(jnp.float32) * fg_col
        pull_over = jnp.sum(pull_c * presf_col)
        k = jnp.sum(presf_col)

        gram = lax.dot_general(cents_t, cents_t, (((0,), (0,)), ((), ())),
                               precision=lax.Precision.HIGHEST,
                               preferred_element_type=jnp.float32)
        n2_row = jnp.sum(cents_t * cents_t, axis=0, keepdims=True)
        n2_col = lax.transpose(n2_row, (1, 0))
        pd2 = jnp.maximum(n2_row + n2_col - 2.0 * gram, 0.0)
        pd = jnp.sqrt(pd2)
        presf_row = lax.transpose(presf_col, (1, 0))
        pairm = presf_col * presf_row * upper
        n_pairs = jnp.sum(pairm)
        push_sum = jnp.sum(jnp.maximum(PUSH_MARGIN - pd, 0.0) * pairm)
        push_term = push_sum / jnp.maximum(n_pairs, 1.0)

        multi = k > 1.0
        contrib = pull_over / jnp.maximum(k, 1.0) + push_term
        total = total + jnp.where(multi, contrib, 0.0)
    out_ref[0, 0] = total


def _loss_from_labels(embt, labels):
    return pl.pallas_call(
        _loss_kernel,
        out_shape=jax.ShapeDtypeStruct((1, 1), jnp.float32),
        out_specs=pl.BlockSpec(memory_space=pltpu.SMEM),
    )(embt, labels)


def kernel(abs_embedding, coordinates, y):
    coords_t = jnp.transpose(coordinates.astype(jnp.int32), (2, 0, 1)).reshape(-1)
    y_flat = y.reshape(-1).astype(jnp.int32)
    labels = _gather_labels_kernel()(coords_t, y_flat)
    loss = _loss_from_labels(abs_embedding, labels)
    return loss[0, 0]


# speedup vs baseline: 1.2701x; 1.2257x over previous
import functools

import jax
import jax.numpy as jnp
from jax import lax
from jax.experimental import pallas as pl
from jax.experimental.pallas import tpu as pltpu
from jax.experimental.pallas import tpu_sc as plsc

PUSH_MARGIN = 1.0
NUM_CLASSES = 8
B, N, C, H, W = 4, 4096, 32, 512, 512
PTS = B * N
NC, NS, L = 2, 16, 16
NW = NC * NS
PER_W = PTS // NW
CHW = 128
NCH = PER_W // CHW


@functools.cache
def _gather_labels_kernel():
    mesh = plsc.VectorSubcoreMesh(
        core_axis_name="c", subcore_axis_name="s", num_cores=NC, num_subcores=NS
    )

    @functools.partial(
        pl.kernel,
        out_type=jax.ShapeDtypeStruct((B, N), jnp.int32),
        mesh=mesh,
        scratch_types=[
            pltpu.VMEM((PER_W,), jnp.int32),
            pltpu.VMEM((PER_W,), jnp.int32),
            pltpu.VMEM((NCH, CHW), jnp.int32),
            pltpu.VMEM((PER_W,), jnp.int32),
            pltpu.SemaphoreType.DMA,
        ],
    )
    def _gather_labels(coords_hbm, y_hbm, out_hbm, cy_v, cx_v, idx_v, lab_v, sem):
        wid = lax.axis_index("s") * NC + lax.axis_index("c")
        base = wid * PER_W
        boff = (base // N) * (H * W)
        cpy = pltpu.async_copy(coords_hbm.at[pl.ds(base, PER_W)], cy_v, sem)
        cpx = pltpu.async_copy(coords_hbm.at[pl.ds(PTS + base, PER_W)], cx_v, sem)
        cpy.wait()
        cpx.wait()
        for j in range(NCH):
            for t in range(CHW // L):
                s = pl.ds(j * CHW + t * L, L)
                idx_v[j, pl.ds(t * L, L)] = cx_v[s] * W + cy_v[s] + boff
        copies = [
            pltpu.async_copy(y_hbm.at[idx_v.at[j]], lab_v.at[pl.ds(j * CHW, CHW)], sem)
            for j in range(NCH)
        ]
        for cp in copies:
            cp.wait()
        pltpu.sync_copy(lab_v, out_hbm.at[base // N, pl.ds(base % N, PER_W)])

    return _gather_labels


def _loss_kernel(embt_ref, lab_ref, out_ref):
    dn_ss = (((1,), (1,)), ((), ()))
    dn_mm = (((1,), (0,)), ((), ()))
    prec = lax.Precision.DEFAULT
    cls_col = lax.broadcasted_iota(jnp.int32, (NUM_CLASSES, 1), 0)
    fg_col = (cls_col >= 1).astype(jnp.float32)
    rr = lax.broadcasted_iota(jnp.int32, (NUM_CLASSES, NUM_CLASSES), 0)
    cc = lax.broadcasted_iota(jnp.int32, (NUM_CLASSES, NUM_CLASSES), 1)
    upper = ((cc > rr) & (rr >= 1)).astype(jnp.float32)

    total = jnp.float32(0.0)
    for b in range(B):
        embt = embt_ref[b].astype(jnp.float32)
        lab = lab_ref[b].reshape(1, N)
        oh = (lab == cls_col).astype(jnp.float32)
        counts_col = jnp.sum(oh, axis=1, keepdims=True)
        safe_col = jnp.maximum(counts_col, 1.0)
        safe_row = lax.transpose(safe_col, (1, 0))
        sums_t = lax.dot_general(embt, oh, dn_ss, precision=prec,
                                 preferred_element_type=jnp.float32)
        cents_t = sums_t / safe_row
        cpp_t = lax.dot_general(cents_t, oh, dn_mm, precision=prec,
                                preferred_element_type=jnp.float32)
        diff = embt - cpp_t
        d = jnp.sqrt(jnp.sum(diff * diff, axis=0, keepdims=True))
        pull_sums = jnp.sum(oh * d, axis=1, keepdims=True)
        pull_c = pull_sums / safe_col
        presf_col = (counts_col > 0.0).astype(jnp.float32) * fg_col
        pull_over = jnp.sum(pull_c * presf_col)
        k = jnp.sum(presf_col)

        gram = lax.dot_general(cents_t, cents_t, (((0,), (0,)), ((), ())),
                               precision=lax.Precision.HIGHEST,
                               preferred_element_type=jnp.float32)
        n2_row = jnp.sum(cents_t * cents_t, axis=0, keepdims=True)
        n2_col = lax.transpose(n2_row, (1, 0))
        pd2 = jnp.maximum(n2_row + n2_col - 2.0 * gram, 0.0)
        pd = jnp.sqrt(pd2)
        presf_row = lax.transpose(presf_col, (1, 0))
        pairm = presf_col * presf_row * upper
        n_pairs = jnp.sum(pairm)
        push_sum = jnp.sum(jnp.maximum(PUSH_MARGIN - pd, 0.0) * pairm)
        push_term = push_sum / jnp.maximum(n_pairs, 1.0)

        multi = k > 1.0
        contrib = pull_over / jnp.maximum(k, 1.0) + push_term
        total = total + jnp.where(multi, contrib, 0.0)
    out_ref[0, 0] = total


def _loss_from_labels(embt, labels):
    return pl.pallas_call(
        _loss_kernel,
        out_shape=jax.ShapeDtypeStruct((1, 1), jnp.float32),
        out_specs=pl.BlockSpec(memory_space=pltpu.SMEM),
    )(embt, labels)


def kernel(abs_embedding, coordinates, y):
    coords_t = jnp.transpose(coordinates.astype(jnp.int32), (2, 0, 1)).reshape(-1)
    y_flat = y.reshape(-1).astype(jnp.int32)
    labels = _gather_labels_kernel()(coords_t, y_flat)
    embt = jnp.transpose(abs_embedding.astype(jnp.bfloat16), (0, 2, 1))
    loss = _loss_from_labels(embt, labels)
    return loss[0, 0]
